# baseline (speedup 1.0000x reference)
import functools

import jax
import jax.numpy as jnp
from jax import lax
from jax.experimental import pallas as pl
from jax.experimental.pallas import tpu as pltpu
from jax.experimental.pallas import tpu_sc as plsc

PED = 1024
R_DIM = 64
M_DIM = 64
BLOCK_ROWS = 64

SC_ROWS = 256
TC_ROWS = PED - SC_ROWS
NW = 32
ROWS_PER_W = SC_ROWS // NW
CHUNK = 256
NCHUNK = PED // CHUNK


def _fused_body(rela_ref, nei_ref, hiddent_ref, w_ref, b_ref, out_ref):
    w = w_ref[...].reshape(1, R_DIM, 1)
    s = jnp.sum(rela_ref[...] * w, axis=1) + b_ref[0]
    mask = nei_ref[...] > 0
    logits = jnp.where(mask, s, jnp.float32(-1e-6))
    logits = jnp.where(logits == 0.0, jnp.float32(-1e-6), logits)
    m = jnp.max(logits, axis=1, keepdims=True)
    e = jnp.exp(logits - m)
    denom = jnp.sum(e, axis=1, keepdims=True)
    p = jnp.where(mask, e / denom, jnp.float32(0.0))
    out_ref[...] = jax.lax.dot_general(
        p, hiddent_ref[...], (((1,), (1,)), ((), ())),
        preferred_element_type=jnp.float32,
    )


def _tc_run(hidden_t, rela_t, nei_index, att_w, att_b):
    n = rela_t.shape[2]
    return pl.pallas_call(
        _fused_body,
        grid=(TC_ROWS // BLOCK_ROWS,),
        in_specs=[
            pl.BlockSpec((BLOCK_ROWS, R_DIM, n), lambda i: (i, 0, 0)),
            pl.BlockSpec((BLOCK_ROWS, n), lambda i: (i, 0)),
            pl.BlockSpec((M_DIM, n), lambda i: (0, 0)),
            pl.BlockSpec((1, R_DIM), lambda i: (0, 0)),
            pl.BlockSpec(memory_space=pltpu.SMEM),
        ],
        out_specs=pl.BlockSpec((BLOCK_ROWS, M_DIM), lambda i: (i, 0)),
        out_shape=jax.ShapeDtypeStruct((TC_ROWS, M_DIM), jnp.float32),
    )(rela_t, nei_index.astype(jnp.int32), hidden_t, att_w, att_b)


def _finish_body(s_ref, nei_ref, hiddent_ref, b_ref, out_ref):
    s = s_ref[...] + b_ref[0]
    mask = nei_ref[...] > 0
    logits = jnp.where(mask, s, jnp.float32(-1e-6))
    logits = jnp.where(logits == 0.0, jnp.float32(-1e-6), logits)
    m = jnp.max(logits, axis=1, keepdims=True)
    e = jnp.exp(logits - m)
    denom = jnp.sum(e, axis=1, keepdims=True)
    p = jnp.where(mask, e / denom, jnp.float32(0.0))
    out_ref[...] = jax.lax.dot_general(
        p, hiddent_ref[...], (((1,), (1,)), ((), ())),
        preferred_element_type=jnp.float32,
    )


def _tc_finish(sc_scores, hidden_t, nei_tail, att_b):
    n = sc_scores.shape[1]
    return pl.pallas_call(
        _finish_body,
        grid=(SC_ROWS // BLOCK_ROWS,),
        in_specs=[
            pl.BlockSpec((BLOCK_ROWS, n), lambda i: (i, 0)),
            pl.BlockSpec((BLOCK_ROWS, n), lambda i: (i, 0)),
            pl.BlockSpec((M_DIM, n), lambda i: (0, 0)),
            pl.BlockSpec(memory_space=pltpu.SMEM),
        ],
        out_specs=pl.BlockSpec((BLOCK_ROWS, M_DIM), lambda i: (i, 0)),
        out_shape=jax.ShapeDtypeStruct((SC_ROWS, M_DIM), jnp.float32),
    )(sc_scores, nei_tail, hidden_t, att_b)


def _sc_body(rela_hbm, wb_hbm, out_hbm, rbuf, wbuf, pbuf, sem, hsem):
    wid = lax.axis_index("s") * 2 + lax.axis_index("c")
    base = TC_ROWS + wid * ROWS_PER_W

    pltpu.async_copy(wb_hbm, wbuf, hsem).wait()

    def start_chunk(row, c, slot):
        pltpu.make_async_copy(
            rela_hbm.at[row, :, pl.ds(c * CHUNK, CHUNK)], rbuf.at[slot], sem.at[slot]
        ).start()

    def wait_chunk(row, c, slot):
        pltpu.make_async_copy(
            rela_hbm.at[row, :, pl.ds(c * CHUNK, CHUNK)], rbuf.at[slot], sem.at[slot]
        ).wait()

    for k in range(ROWS_PER_W):
        row = base + k
        if k == 0:
            start_chunk(row, 0, 0)
        for c in range(NCHUNK):
            slot = (k * NCHUNK + c) % 2
            if c + 1 < NCHUNK:
                start_chunk(row, c + 1, 1 - slot)
            elif k + 1 < ROWS_PER_W:
                start_chunk(row + 1, 0, 1 - slot)
            wait_chunk(row, c, slot)

            def r_step(r, accs, _slot=slot):
                wv = wbuf[pl.ds(r * 16, 16)]
                return tuple(
                    accs[g] + rbuf[_slot, r, pl.ds(g * 16, 16)] * wv
                    for g in range(CHUNK // 16)
                )

            z16 = jnp.zeros((16,), jnp.float32)
            accs = lax.fori_loop(0, R_DIM, r_step, (z16,) * (CHUNK // 16))
            for g in range(CHUNK // 16):
                pbuf[pl.ds(c * CHUNK + g * 16, 16)] = accs[g]

        pltpu.sync_copy(pbuf, out_hbm.at[wid * ROWS_PER_W + k])


@functools.partial(
    pl.kernel,
    mesh=plsc.VectorSubcoreMesh(core_axis_name="c", subcore_axis_name="s"),
    out_type=jax.ShapeDtypeStruct((SC_ROWS, PED), jnp.float32),
    scratch_types=[
        pltpu.VMEM((2, R_DIM, CHUNK), jnp.float32),
        pltpu.VMEM((R_DIM * 16,), jnp.float32),
        pltpu.VMEM((PED,), jnp.float32),
        pltpu.SemaphoreType.DMA((2,)),
        pltpu.SemaphoreType.DMA,
    ],
)
def _sc_run(rela_hbm, wb_hbm, out_hbm, rbuf, wbuf, pbuf, sem, hsem):
    _sc_body(rela_hbm, wb_hbm, out_hbm, rbuf, wbuf, pbuf, sem, hsem)


@jax.jit
def _run(hidden_t, rela_t, nei_index, att_w, att_b):
    wb = jnp.tile(att_w.reshape(R_DIM, 1), (1, 16)).reshape(-1)
    nei32 = nei_index.astype(jnp.int32)
    sc_scores = _sc_run(rela_t, wb)
    tc_out = _tc_run(hidden_t, rela_t, nei32, att_w, att_b)
    sc_out = _tc_finish(sc_scores, hidden_t, nei32[TC_ROWS:], att_b)
    return jnp.concatenate([tc_out, sc_out], axis=0)


def kernel(hidden_state, rela_state, corr_index, nei_index, att_w, att_b):
    del corr_index
    rela_t = jnp.transpose(rela_state, (0, 2, 1))
    hidden_t = jnp.transpose(hidden_state, (1, 0))
    return _run(hidden_t, rela_t, nei_index, att_w, att_b)


# speedup vs baseline: 1.0060x; 1.0060x over previous
import functools

import jax
import jax.numpy as jnp
from jax import lax
from jax.experimental import pallas as pl
from jax.experimental.pallas import tpu as pltpu
from jax.experimental.pallas import tpu_sc as plsc

PED = 1024
R_DIM = 64
M_DIM = 64
BLOCK_ROWS = 64

SC_ROWS = 256
TC_ROWS = PED - SC_ROWS
NW = 32
ROWS_PER_W = SC_ROWS // NW
CHUNK = 256
NCHUNK = PED // CHUNK


def _fused_body(rela_ref, nei_ref, hiddent_ref, w_ref, b_ref, out_ref):
    w = w_ref[...].reshape(1, R_DIM, 1)
    s = jnp.sum(rela_ref[...] * w, axis=1) + b_ref[0]
    mask = nei_ref[...] > 0
    logits = jnp.where(mask, s, jnp.float32(-1e-6))
    logits = jnp.where(logits == 0.0, jnp.float32(-1e-6), logits)
    m = jnp.max(logits, axis=1, keepdims=True)
    e = jnp.exp(logits - m)
    denom = jnp.sum(e, axis=1, keepdims=True)
    p = jnp.where(mask, e / denom, jnp.float32(0.0))
    out_ref[...] = jax.lax.dot_general(
        p, hiddent_ref[...], (((1,), (1,)), ((), ())),
        preferred_element_type=jnp.float32,
    )


def _tc_run(hidden_t, rela_t, nei_index, att_w, att_b):
    n = rela_t.shape[2]
    return pl.pallas_call(
        _fused_body,
        grid=(TC_ROWS // BLOCK_ROWS,),
        in_specs=[
            pl.BlockSpec((BLOCK_ROWS, R_DIM, n), lambda i: (i, 0, 0)),
            pl.BlockSpec((BLOCK_ROWS, n), lambda i: (i, 0)),
            pl.BlockSpec((M_DIM, n), lambda i: (0, 0)),
            pl.BlockSpec((1, R_DIM), lambda i: (0, 0)),
            pl.BlockSpec(memory_space=pltpu.SMEM),
        ],
        out_specs=pl.BlockSpec((BLOCK_ROWS, M_DIM), lambda i: (i, 0)),
        out_shape=jax.ShapeDtypeStruct((TC_ROWS, M_DIM), jnp.float32),
    )(rela_t, nei_index.astype(jnp.int32), hidden_t, att_w, att_b)


def _finish_body(s_ref, nei_ref, hiddent_ref, b_ref, out_ref):
    s = s_ref[...] + b_ref[0]
    mask = nei_ref[...] > 0
    logits = jnp.where(mask, s, jnp.float32(-1e-6))
    logits = jnp.where(logits == 0.0, jnp.float32(-1e-6), logits)
    m = jnp.max(logits, axis=1, keepdims=True)
    e = jnp.exp(logits - m)
    denom = jnp.sum(e, axis=1, keepdims=True)
    p = jnp.where(mask, e / denom, jnp.float32(0.0))
    out_ref[...] = jax.lax.dot_general(
        p, hiddent_ref[...], (((1,), (1,)), ((), ())),
        preferred_element_type=jnp.float32,
    )


def _tc_finish(sc_scores, hidden_t, nei_tail, att_b):
    n = sc_scores.shape[1]
    return pl.pallas_call(
        _finish_body,
        grid=(SC_ROWS // BLOCK_ROWS,),
        in_specs=[
            pl.BlockSpec((BLOCK_ROWS, n), lambda i: (i, 0)),
            pl.BlockSpec((BLOCK_ROWS, n), lambda i: (i, 0)),
            pl.BlockSpec((M_DIM, n), lambda i: (0, 0)),
            pl.BlockSpec(memory_space=pltpu.SMEM),
        ],
        out_specs=pl.BlockSpec((BLOCK_ROWS, M_DIM), lambda i: (i, 0)),
        out_shape=jax.ShapeDtypeStruct((SC_ROWS, M_DIM), jnp.float32),
    )(sc_scores, nei_tail, hidden_t, att_b)


def _sc_body(rela_hbm, wb_hbm, out_hbm, rbuf, wbuf, pbuf, sem, hsem):
    wid = lax.axis_index("s") * 2 + lax.axis_index("c")
    base = TC_ROWS + wid * ROWS_PER_W

    pltpu.async_copy(wb_hbm, wbuf, hsem).wait()

    def start_chunk(row, c, slot):
        pltpu.make_async_copy(
            rela_hbm.at[row, :, pl.ds(c * CHUNK, CHUNK)], rbuf.at[slot], sem.at[slot]
        ).start()

    def wait_chunk(row, c, slot):
        pltpu.make_async_copy(
            rela_hbm.at[row, :, pl.ds(c * CHUNK, CHUNK)], rbuf.at[slot], sem.at[slot]
        ).wait()

    for k in range(ROWS_PER_W):
        row = base + k
        if k == 0:
            start_chunk(row, 0, 0)
        for c in range(NCHUNK):
            slot = (k * NCHUNK + c) % 2
            if c + 1 < NCHUNK:
                start_chunk(row, c + 1, 1 - slot)
            elif k + 1 < ROWS_PER_W:
                start_chunk(row + 1, 0, 1 - slot)
            wait_chunk(row, c, slot)

            def r_step(r, accs, _slot=slot):
                wv = wbuf[pl.ds(r * 16, 16)]
                return tuple(
                    accs[g] + rbuf[_slot, r, pl.ds(g * 16, 16)] * wv
                    for g in range(CHUNK // 16)
                )

            z16 = jnp.zeros((16,), jnp.float32)
            accs = lax.fori_loop(0, R_DIM, r_step, (z16,) * (CHUNK // 16))
            for g in range(CHUNK // 16):
                pbuf[pl.ds(c * CHUNK + g * 16, 16)] = accs[g]

        pltpu.sync_copy(pbuf, out_hbm.at[wid * ROWS_PER_W + k])


@functools.partial(
    pl.kernel,
    mesh=plsc.VectorSubcoreMesh(core_axis_name="c", subcore_axis_name="s"),
    out_type=jax.ShapeDtypeStruct((SC_ROWS, PED), jnp.float32),
    scratch_types=[
        pltpu.VMEM((2, R_DIM, CHUNK), jnp.float32),
        pltpu.VMEM((R_DIM * 16,), jnp.float32),
        pltpu.VMEM((PED,), jnp.float32),
        pltpu.SemaphoreType.DMA((2,)),
        pltpu.SemaphoreType.DMA,
    ],
)
def _sc_run(rela_hbm, wb_hbm, out_hbm, rbuf, wbuf, pbuf, sem, hsem):
    _sc_body(rela_hbm, wb_hbm, out_hbm, rbuf, wbuf, pbuf, sem, hsem)


@jax.jit
def _run(hidden_t, rela_t, nei_index, att_w, att_b):
    wb = jnp.tile(att_w.reshape(R_DIM, 1), (1, 16)).reshape(-1)
    nei32 = nei_index.astype(jnp.int32)
    tc_out = _tc_run(hidden_t, rela_t, nei32, att_w, att_b)
    sc_scores = _sc_run(rela_t, wb)
    sc_out = _tc_finish(sc_scores, hidden_t, nei32[TC_ROWS:], att_b)
    return jnp.concatenate([tc_out, sc_out], axis=0)


def kernel(hidden_state, rela_state, corr_index, nei_index, att_w, att_b):
    del corr_index
    rela_t = jnp.transpose(rela_state, (0, 2, 1))
    hidden_t = jnp.transpose(hidden_state, (1, 0))
    return _run(hidden_t, rela_t, nei_index, att_w, att_b)


# speedup vs baseline: 1.2713x; 1.2637x over previous
import jax
import jax.numpy as jnp
from jax.experimental import pallas as pl
from jax.experimental.pallas import tpu as pltpu

PED = 1024
R_DIM = 64
M_DIM = 64
BLOCK_ROWS = 64


def _fused_body(rela_ref, nei_ref, hiddent_ref, w_ref, b_ref, out_ref):
    w = w_ref[...].reshape(1, R_DIM, 1)
    s = jnp.sum(rela_ref[...] * w, axis=1) + b_ref[0]
    mask = nei_ref[...] > 0
    logits = jnp.where(mask, s, jnp.float32(-1e-6))
    logits = jnp.where(logits == 0.0, jnp.float32(-1e-6), logits)
    m = jnp.max(logits, axis=1, keepdims=True)
    e = jnp.exp(logits - m)
    denom = jnp.sum(e, axis=1, keepdims=True)
    p = jnp.where(mask, e / denom, jnp.float32(0.0))
    out_ref[...] = jax.lax.dot_general(
        p, hiddent_ref[...], (((1,), (1,)), ((), ())),
        preferred_element_type=jnp.float32,
    )


@jax.jit
def _run(hidden_t, rela_t, nei_index, att_w, att_b):
    n = rela_t.shape[0]
    return pl.pallas_call(
        _fused_body,
        grid=(n // BLOCK_ROWS,),
        in_specs=[
            pl.BlockSpec((BLOCK_ROWS, R_DIM, n), lambda i: (i, 0, 0)),
            pl.BlockSpec((BLOCK_ROWS, n), lambda i: (i, 0)),
            pl.BlockSpec((M_DIM, n), lambda i: (0, 0)),
            pl.BlockSpec((1, R_DIM), lambda i: (0, 0)),
            pl.BlockSpec(memory_space=pltpu.SMEM),
        ],
        out_specs=pl.BlockSpec((BLOCK_ROWS, M_DIM), lambda i: (i, 0)),
        out_shape=jax.ShapeDtypeStruct((n, M_DIM), jnp.float32),
    )(rela_t, nei_index.astype(jnp.int32), hidden_t, att_w, att_b)


def kernel(hidden_state, rela_state, corr_index, nei_index, att_w, att_b):
    del corr_index
    rela_t = jnp.transpose(rela_state, (0, 2, 1))
    hidden_t = jnp.transpose(hidden_state, (1, 0))
    return _run(hidden_t, rela_t, nei_index, att_w, att_b)
